# Initial kernel scaffold; baseline (speedup 1.0000x reference)
import jax
import jax.numpy as jnp
from jax.experimental import pallas as pl


def kernel(x, edge_index, W1, b1, Wl, bl, Wr, br, Wo, bo):
    raise NotImplementedError("write your pallas kernel here")


# speedup vs baseline: 21.8561x; 21.8561x over previous
import jax
import jax.numpy as jnp
from jax import lax
from jax.experimental import pallas as pl
from jax.experimental.pallas import tpu as pltpu
from jax.experimental.pallas import tpu_sc as plsc

_CHUNK = 128
_F = 128


def _sc_grid(n, e):
    info = plsc.get_sparse_core_info()
    nc, ns = info.num_cores, info.num_subcores
    rpt = -(-n // (ns * 16)) * 16
    return nc, ns, rpt, rpt * ns, e // _CHUNK


def _sc_counts(col, row, zeros, ones, *, n, e):
    nc, ns, rpt, npad, nchunk = _sc_grid(n, e)
    nw = nc * ns
    iters = -(-nchunk // nw)
    mesh = plsc.VectorSubcoreMesh(core_axis_name="c", subcore_axis_name="s")

    def body(col_hbm, row_hbm, zeros_hbm, ones_hbm, degp_hbm, cntp_hbm,
             idxc_v, idxr_v, ones_v, acc_deg, acc_cnt):
        c = lax.axis_index("c")
        s = lax.axis_index("s")
        wid = s * nc + c
        r0 = s * rpt
        pltpu.sync_copy(zeros_hbm.at[pl.ds(r0, rpt)], acc_deg.at[pl.ds(r0, rpt)])
        pltpu.sync_copy(zeros_hbm.at[pl.ds(r0, rpt)], acc_cnt.at[pl.ds(r0, rpt)])
        pltpu.sync_copy(ones_hbm, ones_v)
        plsc.subcore_barrier()

        @pl.loop(0, iters)
        def _(k):
            chunk = wid + k * nw

            @pl.when(chunk < nchunk)
            def _():
                base = chunk * _CHUNK
                pltpu.sync_copy(col_hbm.at[pl.ds(base, _CHUNK)], idxc_v)
                pltpu.sync_copy(row_hbm.at[pl.ds(base, _CHUNK)], idxr_v)
                pltpu.sync_copy(ones_v, acc_deg.at[idxc_v], add=True)
                pltpu.sync_copy(ones_v, acc_cnt.at[idxr_v], add=True)

        plsc.subcore_barrier()
        o0 = c * npad + r0
        pltpu.sync_copy(acc_deg.at[pl.ds(r0, rpt)], degp_hbm.at[pl.ds(o0, rpt)])
        pltpu.sync_copy(acc_cnt.at[pl.ds(r0, rpt)], cntp_hbm.at[pl.ds(o0, rpt)])

    run = pl.kernel(
        body,
        out_type=(jax.ShapeDtypeStruct((nc * npad,), jnp.float32),
                  jax.ShapeDtypeStruct((nc * npad,), jnp.float32)),
        mesh=mesh,
        scratch_types=(
            pltpu.VMEM((_CHUNK,), jnp.int32),
            pltpu.VMEM((_CHUNK,), jnp.int32),
            pltpu.VMEM((_CHUNK,), jnp.float32),
            pltpu.VMEM_SHARED((npad,), jnp.float32),
            pltpu.VMEM_SHARED((npad,), jnp.float32),
        ),
    )
    return run(col, row, zeros, ones)


def _sc_segsum(src, col, row, zeros, *, n, e):
    nc, ns, rpt, npad, nchunk = _sc_grid(n, e)
    nw = nc * ns
    iters = -(-nchunk // nw)
    mesh = plsc.VectorSubcoreMesh(core_axis_name="c", subcore_axis_name="s")

    def body(src_hbm, col_hbm, row_hbm, zeros_hbm, out_hbm,
             idxg_v, idxs_v, rows_v, acc):
        c = lax.axis_index("c")
        s = lax.axis_index("s")
        wid = s * nc + c
        r0 = s * rpt
        pltpu.sync_copy(zeros_hbm.at[pl.ds(r0, rpt)], acc.at[pl.ds(r0, rpt)])
        plsc.subcore_barrier()

        @pl.loop(0, iters)
        def _(k):
            chunk = wid + k * nw

            @pl.when(chunk < nchunk)
            def _():
                base = chunk * _CHUNK
                pltpu.sync_copy(col_hbm.at[pl.ds(base, _CHUNK)], idxg_v)
                pltpu.sync_copy(row_hbm.at[pl.ds(base, _CHUNK)], idxs_v)
                pltpu.sync_copy(src_hbm.at[idxg_v], rows_v)
                pltpu.sync_copy(rows_v, acc.at[idxs_v], add=True)

        plsc.subcore_barrier()
        pltpu.sync_copy(acc.at[pl.ds(r0, rpt)], out_hbm.at[c, pl.ds(r0, rpt)])

    run = pl.kernel(
        body,
        out_type=jax.ShapeDtypeStruct((nc, npad, _F), jnp.float32),
        mesh=mesh,
        scratch_types=(
            pltpu.VMEM((_CHUNK,), jnp.int32),
            pltpu.VMEM((_CHUNK,), jnp.int32),
            pltpu.VMEM((_CHUNK, _F), jnp.float32),
            pltpu.VMEM_SHARED((npad, _F), jnp.float32),
        ),
    )
    return run(src, col, row, zeros)


def _dot_t(a, w):
    return lax.dot_general(a, w, (((1,), (1,)), ((), ())),
                           preferred_element_type=jnp.float32)


def _mm_body(x_ref, w1_ref, degp_ref, ys_ref):
    deg = degp_ref[0, :, 0:1] + degp_ref[1, :, 0:1] + 1.0
    y = _dot_t(x_ref[...], w1_ref[...])
    ys_ref[...] = y * lax.rsqrt(deg)


def _h_body(degp_ref, aggp_ref, ys_ref, b1_ref, h_ref):
    deg = degp_ref[0, :, 0:1] + degp_ref[1, :, 0:1] + 1.0
    agg = aggp_ref[0] + aggp_ref[1] + ys_ref[...]
    h_ref[...] = jnp.maximum(agg * lax.rsqrt(deg) + b1_ref[...], 0.0)


def _out_body(h_ref, sump_ref, cntp_ref, wl_ref, bl_ref, wr_ref, br_ref,
              wo_ref, bo_ref, o_ref):
    h = h_ref[...]
    cnt = cntp_ref[0, :, 0:1] + cntp_ref[1, :, 0:1]
    mean = (sump_ref[0] + sump_ref[1]) / jnp.maximum(cnt, 1.0)
    t = (_dot_t(h, wl_ref[...]) + bl_ref[...]
         + _dot_t(mean, wr_ref[...]) + br_ref[...])
    t = jnp.maximum(t, 0.0)
    t = t / (jnp.sqrt(jnp.sum(t * t, axis=1, keepdims=True)) + 1e-6)
    z = _dot_t(t, wo_ref[...]) + bo_ref[...]
    m = jnp.max(z, axis=1, keepdims=True)
    ez = jnp.exp(z - m)
    o_ref[...] = ez / jnp.sum(ez, axis=1, keepdims=True)


def kernel(x, edge_index, W1, b1, Wl, bl, Wr, br, Wo, bo):
    n, f_in = x.shape
    e = edge_index.shape[1]
    h1 = W1.shape[0]
    h2 = Wl.shape[0]
    od = Wo.shape[0]
    nc, ns, rpt, npad, _ = _sc_grid(n, e)

    col = edge_index[1]
    row = edge_index[0]
    zeros_1 = jnp.zeros((npad,), jnp.float32)
    ones_1 = jnp.ones((_CHUNK,), jnp.float32)
    zeros_f = jnp.zeros((npad, _F), jnp.float32)
    w1p = jnp.pad(W1, ((0, _F - h1), (0, 0)))
    b1p = jnp.pad(b1, (0, _F - h1)).reshape(1, _F)
    wlp = jnp.pad(Wl, ((0, 0), (0, _F - h1)))
    wrp = jnp.pad(Wr, ((0, 0), (0, _F - h1)))

    degp, cntp = _sc_counts(col, row, zeros_1, ones_1, n=n, e=e)
    degp3 = degp.reshape(nc, npad, 1)
    cntp3 = cntp.reshape(nc, npad, 1)

    bm = 1000
    ys = pl.pallas_call(
        _mm_body,
        grid=(n // bm,),
        in_specs=[
            pl.BlockSpec((bm, f_in), lambda i: (i, 0)),
            pl.BlockSpec((_F, f_in), lambda i: (0, 0)),
            pl.BlockSpec((nc, bm, 1), lambda i: (0, i, 0)),
        ],
        out_specs=pl.BlockSpec((bm, _F), lambda i: (i, 0)),
        out_shape=jax.ShapeDtypeStruct((n, _F), jnp.float32),
    )(x, w1p, degp3)

    aggp = _sc_segsum(ys, col, row, zeros_f, n=n, e=e)

    bm2 = 2000
    h = pl.pallas_call(
        _h_body,
        grid=(n // bm2,),
        in_specs=[
            pl.BlockSpec((nc, bm2, 1), lambda i: (0, i, 0)),
            pl.BlockSpec((nc, bm2, _F), lambda i: (0, i, 0)),
            pl.BlockSpec((bm2, _F), lambda i: (i, 0)),
            pl.BlockSpec((1, _F), lambda i: (0, 0)),
        ],
        out_specs=pl.BlockSpec((bm2, _F), lambda i: (i, 0)),
        out_shape=jax.ShapeDtypeStruct((n, _F), jnp.float32),
    )(degp3, aggp, ys, b1p)

    sump = _sc_segsum(h, col, row, zeros_f, n=n, e=e)

    out = pl.pallas_call(
        _out_body,
        grid=(n // bm2,),
        in_specs=[
            pl.BlockSpec((bm2, _F), lambda i: (i, 0)),
            pl.BlockSpec((nc, bm2, _F), lambda i: (0, i, 0)),
            pl.BlockSpec((nc, bm2, 1), lambda i: (0, i, 0)),
            pl.BlockSpec((h2, _F), lambda i: (0, 0)),
            pl.BlockSpec((1, h2), lambda i: (0, 0)),
            pl.BlockSpec((h2, _F), lambda i: (0, 0)),
            pl.BlockSpec((1, h2), lambda i: (0, 0)),
            pl.BlockSpec((od, h2), lambda i: (0, 0)),
            pl.BlockSpec((1, od), lambda i: (0, 0)),
        ],
        out_specs=pl.BlockSpec((bm2, od), lambda i: (i, 0)),
        out_shape=jax.ShapeDtypeStruct((n, od), jnp.float32),
    )(h, sump, cntp3, wlp, bl.reshape(1, h2), wrp, br.reshape(1, h2),
      Wo, bo.reshape(1, od))
    return out
